# Initial kernel scaffold; baseline (speedup 1.0000x reference)
import jax
import jax.numpy as jnp
from jax.experimental import pallas as pl


def kernel(x, adj, W1, b1, W2, b2):
    raise NotImplementedError("write your pallas kernel here")


# speedup vs baseline: 1.0582x; 1.0582x over previous
import functools

import jax
import jax.numpy as jnp
from jax.experimental import pallas as pl
from jax.experimental.pallas import tpu as pltpu


def _fused_body(x_ref, adj_ref, w1_ref, b1_ref, w2_ref, b2_ref, out_ref, *, tm):
    i = pl.program_id(0)
    agg = jnp.dot(adj_ref[...], x_ref[...], preferred_element_type=jnp.float32)
    h = agg + x_ref[pl.ds(i * tm, tm), :]
    h = jnp.maximum(jnp.dot(h, w1_ref[...], preferred_element_type=jnp.float32)
                    + b1_ref[...], 0.0)
    out_ref[...] = (jnp.dot(h, w2_ref[...], preferred_element_type=jnp.float32)
                    + b2_ref[...])


@functools.partial(jax.jit, static_argnames=())
def _run(x2, adj, W1, b1r, W2, b2r):
    n, d = x2.shape
    tm = 400
    grid = (n // tm,)
    return pl.pallas_call(
        functools.partial(_fused_body, tm=tm),
        grid=grid,
        in_specs=[
            pl.BlockSpec((n, d), lambda i: (0, 0)),
            pl.BlockSpec((tm, n), lambda i: (i, 0)),
            pl.BlockSpec((d, d), lambda i: (0, 0)),
            pl.BlockSpec((1, d), lambda i: (0, 0)),
            pl.BlockSpec((d, d), lambda i: (0, 0)),
            pl.BlockSpec((1, d), lambda i: (0, 0)),
        ],
        out_specs=pl.BlockSpec((tm, d), lambda i: (i, 0)),
        out_shape=jax.ShapeDtypeStruct((n, d), jnp.float32),
        compiler_params=pltpu.CompilerParams(
            dimension_semantics=("arbitrary",),
        ),
    )(x2, adj, W1, b1r, W2, b2r)


def kernel(x, adj, W1, b1, W2, b2):
    if adj.ndim == 3:
        adj = adj[0]
    x2 = x[0]
    out = _run(x2, adj, W1, b1.reshape(1, -1), W2, b2.reshape(1, -1))
    return out[None]
